# Initial kernel scaffold; baseline (speedup 1.0000x reference)
import jax
import jax.numpy as jnp
from jax.experimental import pallas as pl


def kernel(x, edge_index, W1, b1, W2, b2, Wd, bd):
    raise NotImplementedError("write your pallas kernel here")


# speedup vs baseline: 7.8905x; 7.8905x over previous
import functools

import jax
import jax.numpy as jnp
from jax import lax
from jax.experimental import pallas as pl
from jax.experimental.pallas import tpu as pltpu
from jax.experimental.pallas import tpu_sc as plsc

N = 10000
E = 320000
D = 128

NTILE = 16
NSC = 2
NW = NTILE * NSC
NPAD = 10240
ROWS_PER_TILE = NPAD // NTILE
CHUNK = 128
EP_PER_TILE = -(-E // (NW * CHUNK)) * CHUNK
EPAD = EP_PER_TILE * NW
NCHUNK = EP_PER_TILE // CHUNK

BLK = 512
GRID = NPAD // BLK

_mesh = functools.partial(
    plsc.VectorSubcoreMesh, core_axis_name="c", subcore_axis_name="s"
)


def _agg_body(h2_hbm, src_hbm, dst_hbm, zeros_hbm, out_hbm,
              src_v, dst_v, rows_v, acc_sh, sem):
    c = lax.axis_index("c")
    s = lax.axis_index("s")
    w = s * NSC + c
    pltpu.sync_copy(
        zeros_hbm.at[pl.ds(s * ROWS_PER_TILE, ROWS_PER_TILE)],
        acc_sh.at[pl.ds(s * ROWS_PER_TILE, ROWS_PER_TILE)],
    )
    plsc.subcore_barrier()
    base = w * EP_PER_TILE

    def body(g, carry):
        off = pl.multiple_of(base + g * CHUNK, CHUNK)
        pltpu.sync_copy(src_hbm.at[pl.ds(off, CHUNK)], src_v)
        pltpu.sync_copy(dst_hbm.at[pl.ds(off, CHUNK)], dst_v)
        pltpu.async_copy(h2_hbm.at[src_v], rows_v, sem).wait()
        pltpu.sync_copy(rows_v, acc_sh.at[dst_v], add=True)
        return carry

    lax.fori_loop(0, NCHUNK, body, 0)
    plsc.subcore_barrier()
    pltpu.sync_copy(
        acc_sh.at[pl.ds(s * ROWS_PER_TILE, ROWS_PER_TILE)],
        out_hbm.at[c, pl.ds(s * ROWS_PER_TILE, ROWS_PER_TILE)],
    )


_agg_call = functools.partial(
    pl.kernel,
    out_type=jax.ShapeDtypeStruct((NSC, NPAD, D), jnp.float32),
    mesh=_mesh(),
    scratch_types=[
        pltpu.VMEM((CHUNK,), jnp.int32),
        pltpu.VMEM((CHUNK,), jnp.int32),
        pltpu.VMEM((CHUNK, D), jnp.float32),
        pltpu.VMEM_SHARED((NPAD, D), jnp.float32),
        pltpu.SemaphoreType.DMA,
    ],
)(_agg_body)


def _dinv(degp_ref):
    deg = degp_ref[0, :, 0:1] + degp_ref[1, :, 0:1] + 1.0
    return lax.rsqrt(deg)


def _tc1_body(degp_ref, x_ref, w1_ref, h2_ref):
    h = jnp.dot(x_ref[...], w1_ref[...], preferred_element_type=jnp.float32)
    h2_ref[...] = h * _dinv(degp_ref)


def _tc2_body(degp_ref, h2_ref, p_ref, b1_ref, w2_ref, h2b_ref):
    dinv = _dinv(degp_ref)
    acc = h2_ref[...] + p_ref[0] + p_ref[1]
    out1 = jnp.maximum(acc * dinv + b1_ref[...], 0.0)
    h2b_ref[...] = (
        jnp.dot(out1, w2_ref[...], preferred_element_type=jnp.float32) * dinv
    )


def _tc3_body(degp_ref, h2b_ref, q_ref, b2_ref, wd_ref, bd_ref,
              scores_ref, h_ref):
    dinv = _dinv(degp_ref)
    acc = h2b_ref[...] + q_ref[0] + q_ref[1]
    hh = jnp.maximum(acc * dinv + b2_ref[...], 0.0)
    h_ref[...] = hh
    scores_ref[...] = (
        jnp.dot(hh, wd_ref[...], preferred_element_type=jnp.float32)
        + bd_ref[...]
    )


_deg_spec = pl.BlockSpec((NSC, BLK, D), lambda i: (0, i, 0))
_row_spec = pl.BlockSpec((BLK, D), lambda i: (i, 0))
_part_spec = pl.BlockSpec((NSC, BLK, D), lambda i: (0, i, 0))
_mat_spec = pl.BlockSpec((D, D), lambda i: (0, 0))
_vec_spec = pl.BlockSpec((1, D), lambda i: (0, 0))
_row_out = jax.ShapeDtypeStruct((NPAD, D), jnp.float32)


def _tc1(degp, x, W1):
    return pl.pallas_call(
        _tc1_body,
        grid=(GRID,),
        in_specs=[_deg_spec, _row_spec, _mat_spec],
        out_specs=_row_spec,
        out_shape=_row_out,
    )(degp, x, W1)


def _tc2(degp, h2, p, b1, W2):
    return pl.pallas_call(
        _tc2_body,
        grid=(GRID,),
        in_specs=[_deg_spec, _row_spec, _part_spec, _vec_spec, _mat_spec],
        out_specs=_row_spec,
        out_shape=_row_out,
    )(degp, h2, p, b1, W2)


def _tc3(degp, h2b, q, b2, Wd, bd):
    return pl.pallas_call(
        _tc3_body,
        grid=(GRID,),
        in_specs=[_deg_spec, _row_spec, _part_spec, _vec_spec, _mat_spec,
                  _vec_spec],
        out_specs=(_row_spec, _row_spec),
        out_shape=(_row_out, _row_out),
    )(degp, h2b, q, b2, Wd, bd)


def kernel(x, edge_index, W1, b1, W2, b2, Wd, bd):
    x_pad = jnp.zeros((NPAD, D), jnp.float32).at[:N].set(x)
    pad_e = EPAD - E
    src = jnp.concatenate(
        [edge_index[0], jnp.full((pad_e,), N, jnp.int32)])
    dst = jnp.concatenate(
        [edge_index[1], jnp.full((pad_e,), N, jnp.int32)])
    zeros = jnp.zeros((NPAD, D), jnp.float32)
    ones_tab = jnp.ones((NPAD, D), jnp.float32)

    degp = _agg_call(ones_tab, src, dst, zeros)
    h2 = _tc1(degp, x_pad, W1)
    p = _agg_call(h2, src, dst, zeros)
    h2b = _tc2(degp, h2, p, b1.reshape(1, D), W2)
    q = _agg_call(h2b, src, dst, zeros)
    scores_pad, h_pad = _tc3(degp, h2b, q, b2.reshape(1, D), Wd,
                             bd.reshape(1, D))
    return scores_pad[:N], h_pad[:N]


# speedup vs baseline: 8.2873x; 1.0503x over previous
import functools

import jax
import jax.numpy as jnp
from jax import lax
from jax.experimental import pallas as pl
from jax.experimental.pallas import tpu as pltpu
from jax.experimental.pallas import tpu_sc as plsc

N = 10000
E = 320000
D = 128

NTILE = 16
NSC = 2
NW = NTILE * NSC
NPAD = 10240
H = NPAD // 2
ROWS_PER_TILE = NPAD // NTILE
ACC_PER_TILE = H // NTILE
CHUNK = 128
NCHUNK = 80
ACHUNK = 64
ECHUNK = 320
TPAD = 10048
TPIECE = TPAD // 64
EPAD = ECHUNK * ACHUNK * NTILE

BLK = 512
GRID = NPAD // BLK

_mesh = functools.partial(
    plsc.VectorSubcoreMesh, core_axis_name="c", subcore_axis_name="s"
)


def _agg_body(h2_hbm, eidx_hbm, zeros_hbm, out_hbm, idx2, rows0,
              table_sh, acc_sh):
    c = lax.axis_index("c")
    s = lax.axis_index("s")

    def stage_in(i, carry):
        p = i * NTILE + s

        @pl.when(p < TPIECE)
        def _():
            o = pl.multiple_of(p * 64, 64)
            pltpu.sync_copy(h2_hbm.at[pl.ds(o, 64)],
                            table_sh.at[pl.ds(o, 64)])

        return carry

    lax.fori_loop(0, (TPIECE + NTILE - 1) // NTILE, stage_in, 0)

    def stage_zero(i, carry):
        o = s * ACC_PER_TILE + i * 64
        pltpu.sync_copy(zeros_hbm.at[pl.ds(o, 64)],
                        acc_sh.at[pl.ds(o, 64)])
        return carry

    lax.fori_loop(0, ACC_PER_TILE // 64, stage_zero, 0)
    plsc.subcore_barrier()

    def body(g, carry):
        pltpu.sync_copy(eidx_hbm.at[c, s, g], idx2)
        pltpu.sync_copy(table_sh.at[idx2.at[0]], rows0)
        pltpu.sync_copy(rows0, acc_sh.at[idx2.at[1]], add=True)
        return carry

    lax.fori_loop(0, ECHUNK, body, 0)
    plsc.subcore_barrier()

    def stage_out(i, carry):
        o = s * ACC_PER_TILE + i * 64
        pltpu.sync_copy(acc_sh.at[pl.ds(o, 64)],
                        out_hbm.at[c, pl.ds(o, 64)])
        return carry

    lax.fori_loop(0, ACC_PER_TILE // 64, stage_out, 0)


_agg_call = functools.partial(
    pl.kernel,
    out_type=jax.ShapeDtypeStruct((NSC, H, D), jnp.float32),
    mesh=_mesh(),
    scratch_types=[
        pltpu.VMEM((2, ACHUNK), jnp.int32),
        pltpu.VMEM((ACHUNK, D), jnp.float32),
        pltpu.VMEM_SHARED((TPAD, D), jnp.float32),
        pltpu.VMEM_SHARED((H, D), jnp.float32),
    ],
)(_agg_body)


def _deg_body(ones_hbm, dst3_hbm, zeros_hbm, out_hbm,
              dst_all, rows0, deg_sh, ssem):
    c = lax.axis_index("c")
    s = lax.axis_index("s")
    w = s * NSC + c
    pltpu.sync_copy(ones_hbm, rows0)
    pltpu.sync_copy(
        zeros_hbm.at[pl.ds(s * ROWS_PER_TILE, ROWS_PER_TILE)],
        deg_sh.at[pl.ds(s * ROWS_PER_TILE, ROWS_PER_TILE)],
    )
    plsc.subcore_barrier()
    pltpu.sync_copy(dst3_hbm.at[w], dst_all)

    def body(i, carry):
        for j in range(4):
            pltpu.async_copy(rows0, deg_sh.at[dst_all.at[i * 4 + j]], ssem,
                             add=True)
        for j in range(4):
            pltpu.make_async_copy(rows0, deg_sh.at[dst_all.at[i * 4 + j]],
                                  ssem).wait()
        return carry

    lax.fori_loop(0, NCHUNK // 4, body, 0)
    plsc.subcore_barrier()
    pltpu.sync_copy(
        deg_sh.at[pl.ds(s * ROWS_PER_TILE, ROWS_PER_TILE)],
        out_hbm.at[c, pl.ds(s * ROWS_PER_TILE, ROWS_PER_TILE)],
    )


_deg_call = functools.partial(
    pl.kernel,
    out_type=jax.ShapeDtypeStruct((NSC, NPAD, D), jnp.float32),
    mesh=_mesh(),
    scratch_types=[
        pltpu.VMEM((NCHUNK, CHUNK), jnp.int32),
        pltpu.VMEM((CHUNK, D), jnp.float32),
        pltpu.VMEM_SHARED((NPAD, D), jnp.float32),
        pltpu.SemaphoreType.DMA,
    ],
)(_deg_body)


def _dinv(degp_ref):
    deg = degp_ref[0, :, 0:1] + degp_ref[1, :, 0:1] + 1.0
    return lax.rsqrt(deg)


def _tc1_body(degp_ref, x_ref, w1_ref, h2_ref):
    h = jnp.dot(x_ref[...], w1_ref[...], preferred_element_type=jnp.float32)
    h2_ref[...] = h * _dinv(degp_ref)


def _tc2_body(degp_ref, h2_ref, p_ref, b1_ref, w2_ref, h2b_ref):
    dinv = _dinv(degp_ref)
    acc = h2_ref[...] + p_ref[...]
    out1 = jnp.maximum(acc * dinv + b1_ref[...], 0.0)
    h2b_ref[...] = (
        jnp.dot(out1, w2_ref[...], preferred_element_type=jnp.float32) * dinv
    )


def _tc3_body(degp_ref, h2b_ref, q_ref, b2_ref, wd_ref, bd_ref,
              scores_ref, h_ref):
    dinv = _dinv(degp_ref)
    acc = h2b_ref[...] + q_ref[...]
    hh = jnp.maximum(acc * dinv + b2_ref[...], 0.0)
    h_ref[...] = hh
    scores_ref[...] = (
        jnp.dot(hh, wd_ref[...], preferred_element_type=jnp.float32)
        + bd_ref[...]
    )


_deg_spec = pl.BlockSpec((NSC, BLK, D), lambda i: (0, i, 0))
_row_spec = pl.BlockSpec((BLK, D), lambda i: (i, 0))
_mat_spec = pl.BlockSpec((D, D), lambda i: (0, 0))
_vec_spec = pl.BlockSpec((1, D), lambda i: (0, 0))
_row_out = jax.ShapeDtypeStruct((NPAD, D), jnp.float32)


def _tc1(degp, x, W1):
    return pl.pallas_call(
        _tc1_body,
        grid=(GRID,),
        in_specs=[_deg_spec, _row_spec, _mat_spec],
        out_specs=_row_spec,
        out_shape=_row_out,
    )(degp, x, W1)


def _tc2(degp, h2, p, b1, W2):
    return pl.pallas_call(
        _tc2_body,
        grid=(GRID,),
        in_specs=[_deg_spec, _row_spec, _row_spec, _vec_spec, _mat_spec],
        out_specs=_row_spec,
        out_shape=_row_out,
    )(degp, h2, p, b1, W2)


def _tc3(degp, h2b, q, b2, Wd, bd):
    return pl.pallas_call(
        _tc3_body,
        grid=(GRID,),
        in_specs=[_deg_spec, _row_spec, _row_spec, _vec_spec, _mat_spec,
                  _vec_spec],
        out_specs=(_row_spec, _row_spec),
        out_shape=(_row_out, _row_out),
    )(degp, h2b, q, b2, Wd, bd)


def kernel(x, edge_index, W1, b1, W2, b2, Wd, bd):
    x_pad = jnp.zeros((NPAD, D), jnp.float32).at[1:N + 1].set(x)
    pad_e = EPAD - E
    src1 = jnp.concatenate(
        [edge_index[0] + 1, jnp.full((pad_e,), N + 1, jnp.int32)])
    dst1 = jnp.concatenate(
        [edge_index[1] + 1, jnp.full((pad_e,), N + 1, jnp.int32)])
    dst3 = dst1.reshape(NW, NCHUNK, CHUNK)
    in0 = dst1 < H
    srcr = src1.reshape(NTILE, ECHUNK, 1, ACHUNK)
    d0 = jnp.where(in0, dst1, 0).reshape(NTILE, ECHUNK, 1, ACHUNK)
    d1 = jnp.where(in0, H - 1, dst1 - H).reshape(NTILE, ECHUNK, 1, ACHUNK)
    eidx = jnp.stack([
        jnp.concatenate([srcr, d0], axis=2),
        jnp.concatenate([srcr, d1], axis=2),
    ])
    zeros = jnp.zeros((NPAD, D), jnp.float32)
    ones_blk = jnp.ones((CHUNK, D), jnp.float32)

    degp = _deg_call(ones_blk, dst3, zeros)
    h2 = _tc1(degp, x_pad, W1)
    p = _agg_call(h2, eidx, zeros).reshape(NPAD, D)
    h2b = _tc2(degp, h2, p, b1.reshape(1, D), W2)
    q = _agg_call(h2b, eidx, zeros).reshape(NPAD, D)
    scores_pad, h_pad = _tc3(degp, h2b, q, b2.reshape(1, D), Wd,
                             bd.reshape(1, D))
    return scores_pad[1:N + 1], h_pad[1:N + 1]


# speedup vs baseline: 9.8363x; 1.1869x over previous
import functools

import jax
import jax.numpy as jnp
from jax import lax
from jax.experimental import pallas as pl
from jax.experimental.pallas import tpu as pltpu
from jax.experimental.pallas import tpu_sc as plsc

N = 10000
E = 320000
D = 128

NTILE = 16
NSC = 2
NW = NTILE * NSC
NPAD = 10240
H = NPAD // 2
ROWS_PER_TILE = NPAD // NTILE
ACC_PER_TILE = H // NTILE
CHUNK = 128
NCHUNK = 80
ACHUNK = 64
ECHUNK = 320
TPAD = 10048
TPIECE = TPAD // 64
EPAD = ECHUNK * ACHUNK * NTILE

BLK = 512
GRID = NPAD // BLK

_mesh = functools.partial(
    plsc.VectorSubcoreMesh, core_axis_name="c", subcore_axis_name="s"
)


def _agg_body(h2_hbm, eidx_hbm, zeros_hbm, out_hbm, idx2a, idx2b, rows0,
              table_sh, acc_sh, sema, semb):
    c = lax.axis_index("c")
    s = lax.axis_index("s")

    def stage_in(i, carry):
        p = i * NTILE + s

        @pl.when(p < TPIECE)
        def _():
            o = pl.multiple_of(p * 64, 64)
            pltpu.sync_copy(h2_hbm.at[pl.ds(o, 64)],
                            table_sh.at[pl.ds(o, 64)])

        return carry

    lax.fori_loop(0, (TPIECE + NTILE - 1) // NTILE, stage_in, 0)

    def stage_zero(i, carry):
        o = s * ACC_PER_TILE + i * 64
        pltpu.sync_copy(zeros_hbm.at[pl.ds(o, 64)],
                        acc_sh.at[pl.ds(o, 64)])
        return carry

    lax.fori_loop(0, ACC_PER_TILE // 64, stage_zero, 0)
    plsc.subcore_barrier()

    pltpu.async_copy(eidx_hbm.at[c, s, 0], idx2a, sema)

    def body(i, carry):
        ga = i * 2
        gb = ga + 1
        pltpu.make_async_copy(eidx_hbm.at[c, s, ga], idx2a, sema).wait()
        pltpu.async_copy(eidx_hbm.at[c, s, gb], idx2b, semb)
        pltpu.sync_copy(table_sh.at[idx2a.at[0]], rows0)
        pltpu.sync_copy(rows0, acc_sh.at[idx2a.at[1]], add=True)
        pltpu.make_async_copy(eidx_hbm.at[c, s, gb], idx2b, semb).wait()

        @pl.when(gb + 1 <= ECHUNK - 1)
        def _():
            pltpu.async_copy(eidx_hbm.at[c, s, gb + 1], idx2a, sema)

        pltpu.sync_copy(table_sh.at[idx2b.at[0]], rows0)
        pltpu.sync_copy(rows0, acc_sh.at[idx2b.at[1]], add=True)
        return carry

    lax.fori_loop(0, ECHUNK // 2, body, 0)
    plsc.subcore_barrier()

    def stage_out(i, carry):
        o = s * ACC_PER_TILE + i * 64
        pltpu.sync_copy(acc_sh.at[pl.ds(o, 64)],
                        out_hbm.at[c, pl.ds(o, 64)])
        return carry

    lax.fori_loop(0, ACC_PER_TILE // 64, stage_out, 0)


_agg_call = functools.partial(
    pl.kernel,
    out_type=jax.ShapeDtypeStruct((NSC, H, D), jnp.float32),
    mesh=_mesh(),
    scratch_types=[
        pltpu.VMEM((2, ACHUNK), jnp.int32),
        pltpu.VMEM((2, ACHUNK), jnp.int32),
        pltpu.VMEM((ACHUNK, D), jnp.float32),
        pltpu.VMEM_SHARED((TPAD, D), jnp.float32),
        pltpu.VMEM_SHARED((H, D), jnp.float32),
        pltpu.SemaphoreType.DMA,
        pltpu.SemaphoreType.DMA,
    ],
)(_agg_body)


def _deg_body(ones_hbm, dst3_hbm, zeros_hbm, out_hbm,
              dst_all, rows0, deg_sh, ssem):
    c = lax.axis_index("c")
    s = lax.axis_index("s")
    w = s * NSC + c
    pltpu.sync_copy(ones_hbm, rows0)
    pltpu.sync_copy(
        zeros_hbm.at[pl.ds(s * ROWS_PER_TILE, ROWS_PER_TILE)],
        deg_sh.at[pl.ds(s * ROWS_PER_TILE, ROWS_PER_TILE)],
    )
    plsc.subcore_barrier()
    pltpu.sync_copy(dst3_hbm.at[w], dst_all)

    def body(i, carry):
        for j in range(4):
            pltpu.async_copy(rows0, deg_sh.at[dst_all.at[i * 4 + j]], ssem,
                             add=True)
        for j in range(4):
            pltpu.make_async_copy(rows0, deg_sh.at[dst_all.at[i * 4 + j]],
                                  ssem).wait()
        return carry

    lax.fori_loop(0, NCHUNK // 4, body, 0)
    plsc.subcore_barrier()
    pltpu.sync_copy(
        deg_sh.at[pl.ds(s * ROWS_PER_TILE, ROWS_PER_TILE)],
        out_hbm.at[c, pl.ds(s * ROWS_PER_TILE, ROWS_PER_TILE)],
    )


_deg_call = functools.partial(
    pl.kernel,
    out_type=jax.ShapeDtypeStruct((NSC, NPAD, D), jnp.float32),
    mesh=_mesh(),
    scratch_types=[
        pltpu.VMEM((NCHUNK, CHUNK), jnp.int32),
        pltpu.VMEM((CHUNK, D), jnp.float32),
        pltpu.VMEM_SHARED((NPAD, D), jnp.float32),
        pltpu.SemaphoreType.DMA,
    ],
)(_deg_body)


def _dinv(degp_ref):
    deg = degp_ref[0, :, 0:1] + degp_ref[1, :, 0:1] + 1.0
    return lax.rsqrt(deg)


def _tc1_body(degp_ref, x_ref, w1_ref, h2_ref):
    h = jnp.dot(x_ref[...], w1_ref[...], preferred_element_type=jnp.float32)
    h2_ref[...] = h * _dinv(degp_ref)


def _tc2_body(degp_ref, h2_ref, p_ref, b1_ref, w2_ref, h2b_ref):
    dinv = _dinv(degp_ref)
    acc = h2_ref[...] + p_ref[...]
    out1 = jnp.maximum(acc * dinv + b1_ref[...], 0.0)
    h2b_ref[...] = (
        jnp.dot(out1, w2_ref[...], preferred_element_type=jnp.float32) * dinv
    )


def _tc3_body(degp_ref, h2b_ref, q_ref, b2_ref, wd_ref, bd_ref,
              scores_ref, h_ref):
    dinv = _dinv(degp_ref)
    acc = h2b_ref[...] + q_ref[...]
    hh = jnp.maximum(acc * dinv + b2_ref[...], 0.0)
    h_ref[...] = hh
    scores_ref[...] = (
        jnp.dot(hh, wd_ref[...], preferred_element_type=jnp.float32)
        + bd_ref[...]
    )


_deg_spec = pl.BlockSpec((NSC, BLK, D), lambda i: (0, i, 0))
_row_spec = pl.BlockSpec((BLK, D), lambda i: (i, 0))
_mat_spec = pl.BlockSpec((D, D), lambda i: (0, 0))
_vec_spec = pl.BlockSpec((1, D), lambda i: (0, 0))
_row_out = jax.ShapeDtypeStruct((NPAD, D), jnp.float32)


def _tc1(degp, x, W1):
    return pl.pallas_call(
        _tc1_body,
        grid=(GRID,),
        in_specs=[_deg_spec, _row_spec, _mat_spec],
        out_specs=_row_spec,
        out_shape=_row_out,
    )(degp, x, W1)


def _tc2(degp, h2, p, b1, W2):
    return pl.pallas_call(
        _tc2_body,
        grid=(GRID,),
        in_specs=[_deg_spec, _row_spec, _row_spec, _vec_spec, _mat_spec],
        out_specs=_row_spec,
        out_shape=_row_out,
    )(degp, h2, p, b1, W2)


def _tc3(degp, h2b, q, b2, Wd, bd):
    return pl.pallas_call(
        _tc3_body,
        grid=(GRID,),
        in_specs=[_deg_spec, _row_spec, _row_spec, _vec_spec, _mat_spec,
                  _vec_spec],
        out_specs=(_row_spec, _row_spec),
        out_shape=(_row_out, _row_out),
    )(degp, h2b, q, b2, Wd, bd)


def kernel(x, edge_index, W1, b1, W2, b2, Wd, bd):
    x_pad = jnp.zeros((NPAD, D), jnp.float32).at[1:N + 1].set(x)
    pad_e = EPAD - E
    src1 = jnp.concatenate(
        [edge_index[0] + 1, jnp.full((pad_e,), N + 1, jnp.int32)])
    dst1 = jnp.concatenate(
        [edge_index[1] + 1, jnp.full((pad_e,), N + 1, jnp.int32)])
    dst3 = dst1.reshape(NW, NCHUNK, CHUNK)
    in0 = dst1 < H
    srcr = src1.reshape(NTILE, ECHUNK, 1, ACHUNK)
    d0 = jnp.where(in0, dst1, 0).reshape(NTILE, ECHUNK, 1, ACHUNK)
    d1 = jnp.where(in0, H - 1, dst1 - H).reshape(NTILE, ECHUNK, 1, ACHUNK)
    eidx = jnp.stack([
        jnp.concatenate([srcr, d0], axis=2),
        jnp.concatenate([srcr, d1], axis=2),
    ])
    zeros = jnp.zeros((NPAD, D), jnp.float32)
    ones_blk = jnp.ones((CHUNK, D), jnp.float32)

    degp = _deg_call(ones_blk, dst3, zeros)
    h2 = _tc1(degp, x_pad, W1)
    p = _agg_call(h2, eidx, zeros).reshape(NPAD, D)
    h2b = _tc2(degp, h2, p, b1.reshape(1, D), W2)
    q = _agg_call(h2b, eidx, zeros).reshape(NPAD, D)
    scores_pad, h_pad = _tc3(degp, h2b, q, b2.reshape(1, D), Wd,
                             bd.reshape(1, D))
    return scores_pad[1:N + 1], h_pad[1:N + 1]


# speedup vs baseline: 13.0953x; 1.3313x over previous
import functools

import jax
import jax.numpy as jnp
from jax import lax
from jax.experimental import pallas as pl
from jax.experimental.pallas import tpu as pltpu
from jax.experimental.pallas import tpu_sc as plsc

N = 10000
E = 320000
D = 128

NTILE = 16
NSC = 2
NW = NTILE * NSC
NPAD = 10240
H = NPAD // 2
ROWS_PER_TILE = NPAD // NTILE
ACC_PER_TILE = H // NTILE
CHUNK = 128
NCHUNK = 80
ACHUNK = 32
ECHUNK = 640
TPAD = 10048
TPIECE = TPAD // 64
EPAD = ECHUNK * ACHUNK * NTILE

BLK = 512
GRID = NPAD // BLK

_mesh = functools.partial(
    plsc.VectorSubcoreMesh, core_axis_name="c", subcore_axis_name="s"
)


def _agg_body(h2_hbm, eidx_hbm, zeros_hbm, out_hbm, i0, i1, i2, i3,
              rows0, rows1, table_sh, acc_sh,
              is0, is1, is2, is3, ss0, ss1):
    c = lax.axis_index("c")
    s = lax.axis_index("s")

    def stage_in(i, carry):
        p = i * NTILE + s

        @pl.when(p < TPIECE)
        def _():
            o = pl.multiple_of(p * 64, 64)
            pltpu.sync_copy(h2_hbm.at[pl.ds(o, 64)],
                            table_sh.at[pl.ds(o, 64)])

        return carry

    lax.fori_loop(0, (TPIECE + NTILE - 1) // NTILE, stage_in, 0)

    def stage_zero(i, carry):
        o = s * ACC_PER_TILE + i * 64
        pltpu.sync_copy(zeros_hbm.at[pl.ds(o, 64)],
                        acc_sh.at[pl.ds(o, 64)])
        return carry

    lax.fori_loop(0, ACC_PER_TILE // 64, stage_zero, 0)
    plsc.subcore_barrier()

    iq = [i0, i1, i2, i3]
    isem = [is0, is1, is2, is3]
    rows = [rows0, rows1]
    ssem = [ss0, ss1]

    pltpu.async_copy(eidx_hbm.at[c, s, 0], iq[0], isem[0])
    pltpu.async_copy(eidx_hbm.at[c, s, 1], iq[1], isem[1])
    for g in (0, 1):
        pltpu.async_copy(eidx_hbm.at[c, s, g + 2], iq[g + 2], isem[g + 2])
        pltpu.make_async_copy(eidx_hbm.at[c, s, g], iq[g], isem[g]).wait()
        pltpu.sync_copy(table_sh.at[iq[g].at[0]], rows[g])
        pltpu.async_copy(rows[g], acc_sh.at[iq[g].at[1]], ssem[g], add=True)

    def phase(gi, j):
        q = j % 4
        r = j % 2
        q2 = (j + 2) % 4
        pltpu.make_async_copy(rows[r], acc_sh.at[iq[q].at[1]],
                              ssem[r]).wait()
        pltpu.async_copy(eidx_hbm.at[c, s, gi + 2], iq[q2], isem[q2])
        pltpu.make_async_copy(eidx_hbm.at[c, s, gi], iq[q], isem[q]).wait()
        pltpu.sync_copy(table_sh.at[iq[q].at[0]], rows[r])
        pltpu.async_copy(rows[r], acc_sh.at[iq[q].at[1]], ssem[r], add=True)

    def body(i, carry):
        base = 2 + i * 4
        for k in range(4):
            phase(base + k, (2 + k) % 4)
        return carry

    lax.fori_loop(0, (ECHUNK - 4) // 4, body, 0)
    for g in range(ECHUNK - 2, ECHUNK):
        j = g % 4
        q = j % 4
        r = j % 2
        pltpu.make_async_copy(rows[r], acc_sh.at[iq[q].at[1]],
                              ssem[r]).wait()
        pltpu.make_async_copy(eidx_hbm.at[c, s, g], iq[q], isem[q]).wait()
        pltpu.sync_copy(table_sh.at[iq[q].at[0]], rows[r])
        pltpu.async_copy(rows[r], acc_sh.at[iq[q].at[1]], ssem[r], add=True)
    for r in (0, 1):
        g = ECHUNK - 2 + r
        pltpu.make_async_copy(rows[r], acc_sh.at[iq[g % 4].at[1]],
                              ssem[r]).wait()
    plsc.subcore_barrier()

    def stage_out(i, carry):
        o = s * ACC_PER_TILE + i * 64
        pltpu.sync_copy(acc_sh.at[pl.ds(o, 64)],
                        out_hbm.at[c, pl.ds(o, 64)])
        return carry

    lax.fori_loop(0, ACC_PER_TILE // 64, stage_out, 0)


_agg_call = functools.partial(
    pl.kernel,
    out_type=jax.ShapeDtypeStruct((NSC, H, D), jnp.float32),
    mesh=_mesh(),
    scratch_types=(
        [pltpu.VMEM((2, ACHUNK), jnp.int32) for _ in range(4)]
        + [pltpu.VMEM((ACHUNK, D), jnp.float32) for _ in range(2)]
        + [pltpu.VMEM_SHARED((TPAD, D), jnp.float32),
           pltpu.VMEM_SHARED((H, D), jnp.float32)]
        + [pltpu.SemaphoreType.DMA for _ in range(6)]
    ),
)(_agg_body)


def _deg_body(ones_hbm, dst3_hbm, zeros_hbm, out_hbm,
              dst_all, rows0, deg_sh, ssem):
    c = lax.axis_index("c")
    s = lax.axis_index("s")
    w = s * NSC + c
    pltpu.sync_copy(ones_hbm, rows0)
    pltpu.sync_copy(
        zeros_hbm.at[pl.ds(s * ROWS_PER_TILE, ROWS_PER_TILE)],
        deg_sh.at[pl.ds(s * ROWS_PER_TILE, ROWS_PER_TILE)],
    )
    plsc.subcore_barrier()
    pltpu.sync_copy(dst3_hbm.at[w], dst_all)

    def body(i, carry):
        for j in range(4):
            pltpu.async_copy(rows0, deg_sh.at[dst_all.at[i * 4 + j]], ssem,
                             add=True)
        for j in range(4):
            pltpu.make_async_copy(rows0, deg_sh.at[dst_all.at[i * 4 + j]],
                                  ssem).wait()
        return carry

    lax.fori_loop(0, NCHUNK // 4, body, 0)
    plsc.subcore_barrier()
    pltpu.sync_copy(
        deg_sh.at[pl.ds(s * ROWS_PER_TILE, ROWS_PER_TILE)],
        out_hbm.at[c, pl.ds(s * ROWS_PER_TILE, ROWS_PER_TILE)],
    )


_deg_call = functools.partial(
    pl.kernel,
    out_type=jax.ShapeDtypeStruct((NSC, NPAD, D), jnp.float32),
    mesh=_mesh(),
    scratch_types=[
        pltpu.VMEM((NCHUNK, CHUNK), jnp.int32),
        pltpu.VMEM((CHUNK, D), jnp.float32),
        pltpu.VMEM_SHARED((NPAD, D), jnp.float32),
        pltpu.SemaphoreType.DMA,
    ],
)(_deg_body)


def _dinv(degp_ref):
    deg = degp_ref[0, :, 0:1] + degp_ref[1, :, 0:1] + 1.0
    return lax.rsqrt(deg)


def _tc1_body(degp_ref, x_ref, w1_ref, h2_ref):
    h = jnp.dot(x_ref[...], w1_ref[...], preferred_element_type=jnp.float32)
    h2_ref[...] = h * _dinv(degp_ref)


def _tc2_body(degp_ref, h2_ref, p_ref, b1_ref, w2_ref, h2b_ref):
    dinv = _dinv(degp_ref)
    acc = h2_ref[...] + p_ref[...]
    out1 = jnp.maximum(acc * dinv + b1_ref[...], 0.0)
    h2b_ref[...] = (
        jnp.dot(out1, w2_ref[...], preferred_element_type=jnp.float32) * dinv
    )


def _tc3_body(degp_ref, h2b_ref, q_ref, b2_ref, wd_ref, bd_ref,
              scores_ref, h_ref):
    dinv = _dinv(degp_ref)
    acc = h2b_ref[...] + q_ref[...]
    hh = jnp.maximum(acc * dinv + b2_ref[...], 0.0)
    h_ref[...] = hh
    scores_ref[...] = (
        jnp.dot(hh, wd_ref[...], preferred_element_type=jnp.float32)
        + bd_ref[...]
    )


_deg_spec = pl.BlockSpec((NSC, BLK, D), lambda i: (0, i, 0))
_row_spec = pl.BlockSpec((BLK, D), lambda i: (i, 0))
_mat_spec = pl.BlockSpec((D, D), lambda i: (0, 0))
_vec_spec = pl.BlockSpec((1, D), lambda i: (0, 0))
_row_out = jax.ShapeDtypeStruct((NPAD, D), jnp.float32)


def _tc1(degp, x, W1):
    return pl.pallas_call(
        _tc1_body,
        grid=(GRID,),
        in_specs=[_deg_spec, _row_spec, _mat_spec],
        out_specs=_row_spec,
        out_shape=_row_out,
    )(degp, x, W1)


def _tc2(degp, h2, p, b1, W2):
    return pl.pallas_call(
        _tc2_body,
        grid=(GRID,),
        in_specs=[_deg_spec, _row_spec, _row_spec, _vec_spec, _mat_spec],
        out_specs=_row_spec,
        out_shape=_row_out,
    )(degp, h2, p, b1, W2)


def _tc3(degp, h2b, q, b2, Wd, bd):
    return pl.pallas_call(
        _tc3_body,
        grid=(GRID,),
        in_specs=[_deg_spec, _row_spec, _row_spec, _vec_spec, _mat_spec,
                  _vec_spec],
        out_specs=(_row_spec, _row_spec),
        out_shape=(_row_out, _row_out),
    )(degp, h2b, q, b2, Wd, bd)


def kernel(x, edge_index, W1, b1, W2, b2, Wd, bd):
    x_pad = jnp.zeros((NPAD, D), jnp.float32).at[1:N + 1].set(x)
    pad_e = EPAD - E
    src1 = jnp.concatenate(
        [edge_index[0] + 1, jnp.full((pad_e,), N + 1, jnp.int32)])
    dst1 = jnp.concatenate(
        [edge_index[1] + 1, jnp.full((pad_e,), N + 1, jnp.int32)])
    dst3 = dst1.reshape(NW, NCHUNK, CHUNK)
    in0 = dst1 < H
    srcr = src1.reshape(NTILE, ECHUNK, 1, ACHUNK)
    d0 = jnp.where(in0, dst1, 0).reshape(NTILE, ECHUNK, 1, ACHUNK)
    d1 = jnp.where(in0, H - 1, dst1 - H).reshape(NTILE, ECHUNK, 1, ACHUNK)
    eidx = jnp.stack([
        jnp.concatenate([srcr, d0], axis=2),
        jnp.concatenate([srcr, d1], axis=2),
    ])
    zeros = jnp.zeros((NPAD, D), jnp.float32)
    ones_blk = jnp.ones((CHUNK, D), jnp.float32)

    degp = _deg_call(ones_blk, dst3, zeros)
    h2 = _tc1(degp, x_pad, W1)
    p = _agg_call(h2, eidx, zeros).reshape(NPAD, D)
    h2b = _tc2(degp, h2, p, b1.reshape(1, D), W2)
    q = _agg_call(h2b, eidx, zeros).reshape(NPAD, D)
    scores_pad, h_pad = _tc3(degp, h2b, q, b2.reshape(1, D), Wd,
                             bd.reshape(1, D))
    return scores_pad[1:N + 1], h_pad[1:N + 1]
